# baseline (speedup 1.0000x reference)
import functools

import jax
import jax.numpy as jnp
import numpy as np
from jax import lax
from jax.experimental import pallas as pl
from jax.experimental.pallas import tpu as pltpu
from jax.experimental.pallas import tpu_sc as plsc

_W = 64
_C = 14
_N = 1024
_E = 11
_K = _W * _W
_ROWS = _N * _C

_SC_CH = 2
_TC_CH = _C - _SC_CH
_NC, _NS, _L = 2, 16, 16
_NPS = _N // (_NC * _NS)

_COORD = np.array(
    [[12, 12, 8, 7, 12, 9, 10, 2, 1, 3, 4],
     [13, 8, 7, 6, 9, 10, 11, 1, 0, 4, 5]], dtype=np.int32)


def _reduce_body(x_ref, val_ref, xc_ref, yc_ref):
    blk = x_ref[...]
    m = jnp.max(blk, axis=(1, 2))
    w_i = lax.broadcasted_iota(jnp.int32, blk.shape, 1)
    h_i = lax.broadcasted_iota(jnp.int32, blk.shape, 2)
    flat_pos = (w_i * _W + h_i).astype(jnp.float32)
    hit = jnp.where(blk == m[:, None, None, :], flat_pos, float(_K))
    idx = jnp.min(hit, axis=(1, 2))
    val_ref[...] = m[:, None, :]
    xc_ref[...] = ((idx.astype(jnp.int32) % _W).astype(jnp.float32)
                   * (1.0 / _W))[:, None, :]
    yc_ref[...] = (jnp.round(idx * (1.0 / _W)) * (1.0 / _W))[:, None, :]


def _tc_maxpool(x):
    xt = jnp.transpose(x, (1, 2, 3, 0))
    out = jax.ShapeDtypeStruct((_TC_CH, 1, _N), jnp.float32)
    v, xc, yc = pl.pallas_call(
        _reduce_body,
        grid=(_TC_CH,),
        in_specs=[pl.BlockSpec((1, _W, _W, _N), lambda i: (i, 0, 0, 0))],
        out_specs=[pl.BlockSpec((1, 1, _N), lambda i: (i, 0, 0))] * 3,
        out_shape=[out, out, out],
    )(xt)
    return v[:, 0], xc[:, 0], yc[:, 0]


_NV = _K // _L
_MAPS = _SC_CH * _N
_MPS = _MAPS // (_NC * _NS)


def _sc_partial_maxpool(x):
    mesh = plsc.VectorSubcoreMesh(core_axis_name="c", subcore_axis_name="s")
    xr = x.reshape(_N, _C, _K)

    @functools.partial(
        pl.kernel,
        mesh=mesh,
        out_type=[
            jax.ShapeDtypeStruct((_MAPS, _L), jnp.float32),
            jax.ShapeDtypeStruct((_MAPS, _L), jnp.int32),
        ],
        scratch_types=[
            pltpu.VMEM((_K,), jnp.float32),
            pltpu.VMEM((_MPS, _L), jnp.float32),
            pltpu.VMEM((_MPS, _L), jnp.int32),
            pltpu.SemaphoreType.DMA,
            pltpu.SemaphoreType.DMA,
        ],
    )
    def k(x_hbm, pm_hbm, pi_hbm, map_buf, pm_buf, pi_buf, sem_in, sem_out):
        c = lax.axis_index("c")
        s = lax.axis_index("s")
        m0 = (c * _NS + s) * _MPS

        @pl.loop(0, _MPS)
        def _(i):
            mid = m0 + i
            ch = mid // _N
            n = mid % _N
            pltpu.async_copy(
                x_hbm.at[n].at[_TC_CH + ch], map_buf, sem_in).wait()

            def step(t, carry):
                m, idx = carry
                v = map_buf[pl.ds(t * _L, _L)]
                upd = v > m
                m = jnp.maximum(m, v)
                idx = jnp.where(upd, jnp.full((_L,), t, jnp.int32), idx)
                return m, idx

            m, idx = lax.fori_loop(
                0, _NV, step,
                (jnp.full((_L,), -jnp.inf, jnp.float32),
                 jnp.zeros((_L,), jnp.int32)))
            pm_buf[i, :] = m
            pi_buf[i, :] = idx

        pltpu.async_copy(
            pm_buf, pm_hbm.at[pl.ds(m0, _MPS)], sem_out).wait()
        pltpu.async_copy(
            pi_buf, pi_hbm.at[pl.ds(m0, _MPS)], sem_out).wait()

    return k(xr)


def _finish_body(pm_ref, pi_ref, val_ref, xc_ref, yc_ref):
    pm = pm_ref[...]
    pi = pi_ref[...]
    m = jnp.max(pm, axis=1, keepdims=True)
    lane = lax.broadcasted_iota(jnp.int32, pm.shape, 1)
    flat = (pi * _L + lane).astype(jnp.float32)
    hit = jnp.where(pm == m, flat, float(_K))
    idx = jnp.min(hit, axis=1, keepdims=True)
    val_ref[...] = jnp.broadcast_to(m, pm.shape)
    xc_ref[...] = jnp.broadcast_to(
        (idx.astype(jnp.int32) % _W).astype(jnp.float32) * (1.0 / _W),
        pm.shape)
    yc_ref[...] = jnp.broadcast_to(
        jnp.round(idx * (1.0 / _W)) * (1.0 / _W), pm.shape)


def _sc_maxpool(x):
    pm, pi = _sc_partial_maxpool(x)
    out = jax.ShapeDtypeStruct((_MAPS, _L), jnp.float32)
    v, xc, yc = pl.pallas_call(
        _finish_body,
        in_specs=[pl.BlockSpec((_MAPS, _L), lambda: (0, 0))] * 2,
        out_specs=[pl.BlockSpec((_MAPS, _L), lambda: (0, 0))] * 3,
        out_shape=[out, out, out],
    )(pm, pi)
    return (v[:, 0].reshape(_SC_CH, _N),
            xc[:, 0].reshape(_SC_CH, _N),
            yc[:, 0].reshape(_SC_CH, _N))


def kernel(x):
    v_tc, xc_tc, yc_tc = _tc_maxpool(x)
    v_sc, xc_sc, yc_sc = _sc_maxpool(x)
    v = jnp.concatenate([v_tc, v_sc], axis=0)
    xc = jnp.concatenate([xc_tc, xc_sc], axis=0)
    yc = jnp.concatenate([yc_tc, yc_sc], axis=0)
    feature = jnp.stack([v, xc, yc], axis=-1)
    feature = jnp.transpose(feature, (1, 0, 2)).reshape(_ROWS, 3)
    coord = jnp.asarray(_COORD)
    offsets = jnp.arange(_N, dtype=jnp.int32) * _C
    edge_index = (coord[:, None, :] + offsets[None, :, None]).reshape(2, _N * _E)
    return feature, edge_index


# speedup vs baseline: 5.1831x; 5.1831x over previous
import functools

import jax
import jax.numpy as jnp
import numpy as np
from jax import lax
from jax.experimental import pallas as pl

_W = 64
_C = 14
_N = 1024
_E = 11
_ROWS = _N * _C
_K = _W * _W
_COORD_PAD = np.zeros((2, 16), dtype=np.int32)
_COORD_PAD[:, :_E] = np.array(
    [[12, 12, 8, 7, 12, 9, 10, 2, 1, 3, 4],
     [13, 8, 7, 6, 9, 10, 11, 1, 0, 4, 5]], dtype=np.int32)

_NSTREAM = 4
_NB = _N // _NSTREAM


def _reduce_body(*refs):
    x_refs, (val_ref, xc_ref, yc_ref) = refs[:_NSTREAM], refs[_NSTREAM:]
    for s, x_ref in enumerate(x_refs):
        blk = x_ref[...]
        m = jnp.max(blk, axis=(1, 2))
        w_i = lax.broadcasted_iota(jnp.int32, blk.shape, 1)
        h_i = lax.broadcasted_iota(jnp.int32, blk.shape, 2)
        flat_pos = (w_i * _W + h_i).astype(jnp.float32)
        hit = jnp.where(blk == m[:, None, None, :], flat_pos, float(_K))
        idx = jnp.min(hit, axis=(1, 2))
        sl = pl.ds(s * _NB, _NB)
        val_ref[:, :, sl] = m[:, None, :]
        xc_ref[:, :, sl] = ((idx.astype(jnp.int32) % _W).astype(jnp.float32)
                            * (1.0 / _W))[:, None, :]
        yc_ref[:, :, sl] = (jnp.round(idx * (1.0 / _W)) * (1.0 / _W))[:, None, :]


def _maxpool_keypoints(x):
    xt = jnp.transpose(x, (1, 2, 3, 0))
    out = jax.ShapeDtypeStruct((_C, 1, _N), jnp.float32)
    v, xc, yc = pl.pallas_call(
        _reduce_body,
        grid=(_C,),
        in_specs=[
            pl.BlockSpec((1, _W, _W, _NB),
                         functools.partial(lambda s, i: (i, 0, 0, s), s))
            for s in range(_NSTREAM)
        ],
        out_specs=[pl.BlockSpec((1, 1, _N), lambda i: (i, 0, 0))] * 3,
        out_shape=[out, out, out],
    )(*([xt] * _NSTREAM))
    return v[:, 0], xc[:, 0], yc[:, 0]


def kernel(x):
    value, xc, yc = _maxpool_keypoints(x)
    feature = jnp.stack([value, xc, yc], axis=-1)
    feature = jnp.transpose(feature, (1, 0, 2)).reshape(_ROWS, 3)
    coord = jnp.asarray(_COORD_PAD[:, :_E])
    offsets = jnp.arange(_N, dtype=jnp.int32) * _C
    edge_index = (coord[:, None, :] + offsets[None, :, None]).reshape(2, _N * _E)
    return feature, edge_index
